# baseline (speedup 1.0000x reference)
import numpy as np
import jax
import jax.numpy as jnp
from jax import lax
from jax.experimental import pallas as pl
from jax.experimental.pallas import tpu as pltpu

_P = 1024
_G = 32
_RPB = 16


def _build_bilinear_matrix():
    g = _G
    c = np.arange(g * g)
    i = c // g
    j = c % g
    y = i.astype(np.float64) / g * (g - 1)
    x = j.astype(np.float64) / g * (g - 1)
    y0 = np.clip(np.floor(y).astype(np.int64), 0, g - 1)
    x0 = np.clip(np.floor(x).astype(np.int64), 0, g - 1)
    y1 = np.clip(y0 + 1, 0, g - 1)
    x1 = np.clip(x0 + 1, 0, g - 1)
    wy = y - y0
    wx = x - x0
    A = np.zeros((g * g, g * g), dtype=np.float64)
    np.add.at(A, (c, y0 * g + x0), (1 - wy) * (1 - wx))
    np.add.at(A, (c, y0 * g + x1), (1 - wy) * wx)
    np.add.at(A, (c, y1 * g + x0), wy * (1 - wx))
    np.add.at(A, (c, y1 * g + x1), wy * wx)
    return A.astype(np.float32)


def _build_constants(H):
    g = _G
    gr = g * _RPB
    bh = H // g
    pr = np.arange(gr * (bh // 2))
    U8T = (pr[None, :] // (bh // 2) == np.arange(gr)[:, None])
    p = np.arange(H)
    U = (p[:, None] // bh == np.arange(g)[None, :])
    a = np.arange(g)
    TRIU = (a[:, None] <= a[None, :])
    rows = np.arange(gr)
    same = rows[:, None] // g == rows[None, :] // g
    SL4 = same & (rows[None, :] < rows[:, None])
    BD = same
    q = np.arange(4 * g * g)
    r128 = np.arange(4 * g)
    Rep4 = (q[None, :] // g == r128[:, None])
    D4 = (q[None, :] % g == a[:, None])
    return (
        jnp.asarray(U8T, dtype=jnp.bfloat16),
        jnp.asarray(U, dtype=jnp.bfloat16),
        jnp.asarray(TRIU, dtype=jnp.float32),
        jnp.asarray(SL4, dtype=jnp.float32),
        jnp.asarray(BD, dtype=jnp.float32),
        jnp.asarray(Rep4, dtype=jnp.float32),
        jnp.asarray(D4, dtype=jnp.float32),
        jnp.asarray(_build_bilinear_matrix()),
    )


def _region_pool_kernel(mask_ref, fmap_ref, u8t_ref, u_ref, triu_ref, sl4_ref,
                        bd_ref, rep4_ref, d4_ref, a_ref, out_ref, wall_ref):
    g = _G
    gr = g * _RPB
    R = wall_ref.shape[0]
    rb = pl.program_id(1)
    nb = R // _RPB

    mi = mask_ref[0, 0]
    f8 = mi[:, 0:8, :] + mi[:, 8:16, :]
    m2d = f8.reshape(gr * 8, f8.shape[2]).astype(jnp.bfloat16)
    rowred = jnp.dot(u8t_ref[...], m2d,
                     preferred_element_type=jnp.float32)
    cnt = jnp.dot(rowred.astype(jnp.bfloat16), u_ref[...],
                  preferred_element_type=jnp.float32)
    occ = cnt > 0.5

    ri = lax.broadcasted_iota(jnp.int32, (gr, g), 0)
    ci = lax.broadcasted_iota(jnp.int32, (gr, g), 1)
    pre = jnp.sum(occ.astype(jnp.float32), axis=1, keepdims=True)
    nr = jnp.dot(bd_ref[...], pre, preferred_element_type=jnp.float32)
    occ = occ | ((ri % g == 0) & (ci == 0) & (nr < 0.5))
    o = occ.astype(jnp.float32)

    crow = jnp.dot(o, triu_ref[...], preferred_element_type=jnp.float32)
    rowsum = jnp.sum(o, axis=1, keepdims=True)
    prefix = jnp.dot(sl4_ref[...], rowsum,
                     preferred_element_type=jnp.float32)
    rank = prefix + crow - 1.0

    n = jnp.dot(bd_ref[...], rowsum, preferred_element_type=jnp.float32)
    qd = jnp.floor(float(_P) / n)
    rem = float(_P) - qd * n
    w = o * (qd + (rank < rem).astype(jnp.float32))

    for h in range(_RPB // 4):
        wh = w[h * 128:(h + 1) * 128, :]
        wb = lax.dot_general(wh, rep4_ref[...], (((0,), (0,)), ((), ())),
                             preferred_element_type=jnp.float32)
        wcat = jnp.sum(wb * d4_ref[...], axis=0, keepdims=True)
        for k in range(4):
            wall_ref[pl.ds(rb * _RPB + h * 4 + k, 1), :] = \
                wcat[:, k * g * g:(k + 1) * g * g]

    @pl.when(rb == nb - 1)
    def _():
        vall = jnp.dot(wall_ref[...], a_ref[...],
                       preferred_element_type=jnp.float32)
        out = jnp.dot(vall, fmap_ref[0],
                      preferred_element_type=jnp.float32)
        out_ref[0, :, 0, :] = out * (1.0 / float(_P))


def _make_call(B, R, H, W, HW, C, interpret=False):
    full = lambda shape: pl.BlockSpec(shape, lambda b, r: (0,) * len(shape))
    gr = _G * _RPB
    return pl.pallas_call(
        _region_pool_kernel,
        grid=(B, R // _RPB),
        in_specs=[
            pl.BlockSpec((1, 1, gr, H // _G, W),
                         lambda b, r: (b, r, 0, 0, 0)),
            pl.BlockSpec((1, HW, C), lambda b, r: (b, 0, 0)),
            full((gr, gr * 8)),
            full((H, _G)),
            full((_G, _G)),
            full((gr, gr)),
            full((gr, gr)),
            full((4 * _G, 4 * _G * _G)),
            full((_G, 4 * _G * _G)),
            full((_G * _G, _G * _G)),
        ],
        out_specs=pl.BlockSpec((1, R, 1, C), lambda b, r: (b, 0, 0, 0)),
        out_shape=jax.ShapeDtypeStruct((B, R, 1, C), jnp.float32),
        scratch_shapes=[pltpu.VMEM((R, _G * _G), jnp.float32)],
        compiler_params=pltpu.CompilerParams(
            dimension_semantics=("parallel", "arbitrary")),
        interpret=interpret,
    )


def kernel(feature_map, region_masks):
    B, HW, C = feature_map.shape
    _, R, H, W = region_masks.shape
    consts = _build_constants(H)
    call = _make_call(B, R, H, W, HW, C)
    masks5 = region_masks.reshape(B, R // _RPB, _RPB * _G, H // _G, W)
    return call(masks5, feature_map, *consts)


# speedup vs baseline: 1.0130x; 1.0130x over previous
import numpy as np
import jax
import jax.numpy as jnp
from jax import lax
from jax.experimental import pallas as pl
from jax.experimental.pallas import tpu as pltpu

_P = 1024
_G = 32
_RPB = 8


def _build_bilinear_matrix():
    g = _G
    c = np.arange(g * g)
    i = c // g
    j = c % g
    y = i.astype(np.float64) / g * (g - 1)
    x = j.astype(np.float64) / g * (g - 1)
    y0 = np.clip(np.floor(y).astype(np.int64), 0, g - 1)
    x0 = np.clip(np.floor(x).astype(np.int64), 0, g - 1)
    y1 = np.clip(y0 + 1, 0, g - 1)
    x1 = np.clip(x0 + 1, 0, g - 1)
    wy = y - y0
    wx = x - x0
    A = np.zeros((g * g, g * g), dtype=np.float64)
    np.add.at(A, (c, y0 * g + x0), (1 - wy) * (1 - wx))
    np.add.at(A, (c, y0 * g + x1), (1 - wy) * wx)
    np.add.at(A, (c, y1 * g + x0), wy * (1 - wx))
    np.add.at(A, (c, y1 * g + x1), wy * wx)
    return A.astype(np.float32)


def _build_constants(H):
    g = _G
    gr = g * _RPB
    bh = H // g
    pr = np.arange(gr * (bh // 2))
    U8T = (pr[None, :] // (bh // 2) == np.arange(gr)[:, None])
    p = np.arange(H)
    U = (p[:, None] // bh == np.arange(g)[None, :])
    a = np.arange(g)
    TRIU = (a[:, None] <= a[None, :])
    rows = np.arange(gr)
    same = rows[:, None] // g == rows[None, :] // g
    SL4 = same & (rows[None, :] < rows[:, None])
    BD = same
    q = np.arange(4 * g * g)
    r128 = np.arange(4 * g)
    Rep4 = (q[None, :] // g == r128[:, None])
    D4 = (q[None, :] % g == a[:, None])
    return (
        jnp.asarray(U8T, dtype=jnp.bfloat16),
        jnp.asarray(U, dtype=jnp.bfloat16),
        jnp.asarray(TRIU, dtype=jnp.float32),
        jnp.asarray(SL4, dtype=jnp.float32),
        jnp.asarray(BD, dtype=jnp.float32),
        jnp.asarray(Rep4, dtype=jnp.float32),
        jnp.asarray(D4, dtype=jnp.float32),
        jnp.asarray(_build_bilinear_matrix()),
    )


def _region_pool_kernel(mask_ref, fmap_ref, u8t_ref, u_ref, triu_ref, sl4_ref,
                        bd_ref, rep4_ref, d4_ref, a_ref, out_ref, wall_ref):
    g = _G
    gr = g * _RPB
    R = wall_ref.shape[0]
    rb = pl.program_id(1)
    nb = R // _RPB

    mi = mask_ref[0, 0]
    f8 = mi[:, 0:8, :] + mi[:, 8:16, :]
    m2d = f8.reshape(gr * 8, f8.shape[2]).astype(jnp.bfloat16)
    rowred = jnp.dot(u8t_ref[...], m2d,
                     preferred_element_type=jnp.float32)
    cnt = jnp.dot(rowred.astype(jnp.bfloat16), u_ref[...],
                  preferred_element_type=jnp.float32)
    occ = cnt > 0.5

    ri = lax.broadcasted_iota(jnp.int32, (gr, g), 0)
    ci = lax.broadcasted_iota(jnp.int32, (gr, g), 1)
    pre = jnp.sum(occ.astype(jnp.float32), axis=1, keepdims=True)
    nr = jnp.dot(bd_ref[...], pre, preferred_element_type=jnp.float32)
    occ = occ | ((ri % g == 0) & (ci == 0) & (nr < 0.5))
    o = occ.astype(jnp.float32)

    crow = jnp.dot(o, triu_ref[...], preferred_element_type=jnp.float32)
    rowsum = jnp.sum(o, axis=1, keepdims=True)
    prefix = jnp.dot(sl4_ref[...], rowsum,
                     preferred_element_type=jnp.float32)
    rank = prefix + crow - 1.0

    n = jnp.dot(bd_ref[...], rowsum, preferred_element_type=jnp.float32)
    qd = jnp.floor(float(_P) / n)
    rem = float(_P) - qd * n
    w = o * (qd + (rank < rem).astype(jnp.float32))

    for h in range(_RPB // 4):
        wh = w[h * 128:(h + 1) * 128, :]
        wb = lax.dot_general(wh, rep4_ref[...], (((0,), (0,)), ((), ())),
                             preferred_element_type=jnp.float32)
        wcat = jnp.sum(wb * d4_ref[...], axis=0, keepdims=True)
        for k in range(4):
            wall_ref[pl.ds(rb * _RPB + h * 4 + k, 1), :] = \
                wcat[:, k * g * g:(k + 1) * g * g]

    @pl.when(rb == nb - 1)
    def _():
        vall = jnp.dot(wall_ref[...], a_ref[...],
                       preferred_element_type=jnp.float32)
        out = jnp.dot(vall, fmap_ref[0],
                      preferred_element_type=jnp.float32)
        out_ref[0, :, 0, :] = out * (1.0 / float(_P))


def _make_call(B, R, H, W, HW, C):
    full = lambda shape: pl.BlockSpec(shape, lambda b, r: (0,) * len(shape))
    gr = _G * _RPB
    return pl.pallas_call(
        _region_pool_kernel,
        grid=(B, R // _RPB),
        in_specs=[
            pl.BlockSpec((1, 1, gr, H // _G, W),
                         lambda b, r: (b, r, 0, 0, 0)),
            pl.BlockSpec((1, HW, C), lambda b, r: (b, 0, 0)),
            full((gr, gr * 8)),
            full((H, _G)),
            full((_G, _G)),
            full((gr, gr)),
            full((gr, gr)),
            full((4 * _G, 4 * _G * _G)),
            full((_G, 4 * _G * _G)),
            full((_G * _G, _G * _G)),
        ],
        out_specs=pl.BlockSpec((1, R, 1, C), lambda b, r: (b, 0, 0, 0)),
        out_shape=jax.ShapeDtypeStruct((B, R, 1, C), jnp.float32),
        scratch_shapes=[pltpu.VMEM((R, _G * _G), jnp.float32)],
        compiler_params=pltpu.CompilerParams(
            dimension_semantics=("parallel", "arbitrary")),
    )


def kernel(feature_map, region_masks):
    B, HW, C = feature_map.shape
    _, R, H, W = region_masks.shape
    consts = _build_constants(H)
    call = _make_call(B, R, H, W, HW, C)
    masks5 = region_masks.reshape(B, R // _RPB, _RPB * _G, H // _G, W)
    return call(masks5, feature_map, *consts)
